# Initial kernel scaffold; baseline (speedup 1.0000x reference)
import jax
import jax.numpy as jnp
from jax.experimental import pallas as pl


def kernel(q, k, v, Wq, Wk, Wv, Wo):
    raise NotImplementedError("write your pallas kernel here")


# speedup vs baseline: 3.7218x; 3.7218x over previous
import functools

import jax
import jax.numpy as jnp
from jax.experimental import pallas as pl
from jax.experimental.pallas import tpu as pltpu

H = 12
DH = 64
D = H * DH
SPAN = 32
L = 2048
BS = 256
NB = L // BS
SCALE = 1.0 / (DH ** 0.5)
NEG = -jnp.inf


def _proj_body(q_ref, k_ref, v_ref, wq_ref, wk_ref, wv_ref,
               q_out, k_out, v_out):
    q_out[...] = jnp.dot(q_ref[...], wq_ref[...],
                         preferred_element_type=jnp.float32)
    k_out[...] = jnp.dot(k_ref[...], wk_ref[...],
                         preferred_element_type=jnp.float32)
    v_out[...] = jnp.dot(v_ref[...], wv_ref[...],
                         preferred_element_type=jnp.float32)


def _score_body(q_ref, kcur_ref, kprev_ref, s_ref):
    jb = pl.program_id(0)
    qb = q_ref[...]
    kwin = jnp.concatenate([kprev_ref[BS - (SPAN - 1):, :], kcur_ref[...]],
                           axis=0)
    rows = jax.lax.broadcasted_iota(jnp.int32, (BS, H), 0)
    base = jb * BS + rows - (SPAN - 1)
    for d in range(SPAN):
        x = qb * kwin[d:d + BS, :]
        w2 = D
        while w2 > H:
            w2 //= 2
            x = x[:, :w2] + x[:, w2:]
        s = x * SCALE
        s = jnp.where(base + d >= 0, s, NEG)
        s_ref[:, d * H:(d + 1) * H] = s


def _out_body(s_ref, vcur_ref, vprev_ref, wo_ref, o_ref, mz_ref):
    jb = pl.program_id(0)

    @pl.when(jb == 0)
    def _():
        s_all = s_ref[...]
        m = jnp.max(s_all, axis=0, keepdims=True)
        z = jnp.sum(jnp.exp(s_all - m), axis=0, keepdims=True)
        mz_ref[0:1, :] = m
        mz_ref[1:2, :] = z

    m = mz_ref[0:1, :]
    z = mz_ref[1:2, :]
    s_blk = s_ref[pl.ds(jb * BS, BS), :]
    w = jnp.exp(s_blk - m) / z
    vwin = jnp.concatenate([vprev_ref[BS - (SPAN - 1):, :], vcur_ref[...]],
                           axis=0)
    acc = jnp.zeros((BS, D), dtype=jnp.float32)
    for d in range(SPAN):
        wd = w[:, d * H:(d + 1) * H]
        we = jax.lax.broadcast_in_dim(wd, (BS, DH, H), (0, 2)).reshape(BS, D)
        acc = acc + we * vwin[d:d + BS, :]
    o_ref[...] = jnp.dot(acc, wo_ref[...],
                         preferred_element_type=jnp.float32)


@jax.jit
def kernel(q, k, v, Wq, Wk, Wv, Wo):
    qs = q[0]
    ks = k[0]
    vs = v[0]
    perm = (jnp.arange(D) % H) * DH + jnp.arange(D) // H
    wqT = Wq.T[:, perm]
    wkT = Wk.T[:, perm]
    wvT = Wv.T[:, perm]
    woT = Wo.T[perm, :]

    blk = pl.BlockSpec((BS, D), lambda i: (i, 0))
    full_w = pl.BlockSpec((D, D), lambda i: (0, 0))
    prev = pl.BlockSpec((BS, D), lambda i: (jnp.maximum(i - 1, 0), 0))

    Q, K, V = pl.pallas_call(
        _proj_body,
        grid=(NB,),
        in_specs=[blk, blk, blk, full_w, full_w, full_w],
        out_specs=[blk, blk, blk],
        out_shape=[jax.ShapeDtypeStruct((L, D), jnp.float32)] * 3,
    )(qs, ks, vs, wqT, wkT, wvT)

    s_blk = pl.BlockSpec((BS, SPAN * H), lambda i: (i, 0))
    S = pl.pallas_call(
        _score_body,
        grid=(NB,),
        in_specs=[blk, blk, prev],
        out_specs=s_blk,
        out_shape=jax.ShapeDtypeStruct((L, SPAN * H), jnp.float32),
    )(Q, K, K)

    s_full = pl.BlockSpec((L, SPAN * H), lambda i: (0, 0))
    out2d = pl.pallas_call(
        _out_body,
        grid=(NB,),
        in_specs=[s_full, blk, prev, full_w],
        out_specs=blk,
        out_shape=jax.ShapeDtypeStruct((L, D), jnp.float32),
        scratch_shapes=[pltpu.VMEM((8, SPAN * H), jnp.float32)],
    )(S, V, V, woT)

    return out2d[None]


# speedup vs baseline: 11.3385x; 3.0465x over previous
import functools

import jax
import jax.numpy as jnp
from jax.experimental import pallas as pl
from jax.experimental.pallas import tpu as pltpu

H = 12
DH = 64
D = H * DH
SPAN = 32
L = 2048
BS = 256
NB = L // BS
SCALE = 1.0 / (DH ** 0.5)
NEG = -jnp.inf


def _proj_body(q_ref, k_ref, v_ref, wq_ref, wk_ref, wv_ref,
               q_out, k_out, v_out):
    q_out[...] = jnp.dot(q_ref[...], wq_ref[...],
                         preferred_element_type=jnp.float32)
    k_out[...] = jnp.dot(k_ref[...], wk_ref[...],
                         preferred_element_type=jnp.float32)
    v_out[...] = jnp.dot(v_ref[...], wv_ref[...],
                         preferred_element_type=jnp.float32)


DG = 4
W2 = 192


def _score_body(q_ref, kcur_ref, kprev_ref, s_ref):
    jb = pl.program_id(0)
    qb = q_ref[...]
    kwin = jnp.concatenate([kprev_ref[BS - (SPAN - 1):, :], kcur_ref[...]],
                           axis=0)
    rr = jax.lax.broadcasted_iota(jnp.int32, (DG * W2, DG * H), 0)
    cc = jax.lax.broadcasted_iota(jnp.int32, (DG * W2, DG * H), 1)
    rmat = jnp.where((rr % H == cc % H) & (rr // W2 == cc // H),
                     1.0, 0.0).astype(jnp.float32)
    rows = jax.lax.broadcasted_iota(jnp.int32, (BS, DG * H), 0)
    dcol = jax.lax.broadcasted_iota(jnp.int32, (BS, DG * H), 1) // H
    for d0 in range(0, SPAN, DG):
        parts = []
        for d in range(d0, d0 + DG):
            x = qb * kwin[d:d + BS, :]
            x = x[:, :D // 2] + x[:, D // 2:]
            x = x[:, :W2] + x[:, W2:]
            parts.append(x)
        s4 = jnp.dot(jnp.concatenate(parts, axis=1), rmat,
                     preferred_element_type=jnp.float32)
        s4 = jnp.where(jb * BS + rows + (d0 + dcol) - (SPAN - 1) >= 0,
                       s4, NEG)
        s_ref[:, d0 * H:(d0 + DG) * H] = s4


def _out_body(s_ref, vcur_ref, vprev_ref, wo_ref, o_ref, mz_ref):
    jb = pl.program_id(0)

    @pl.when(jb == 0)
    def _():
        s_all = s_ref[...]
        m = jnp.max(s_all, axis=0, keepdims=True)
        z = jnp.sum(jnp.exp(s_all - m), axis=0, keepdims=True)
        mz_ref[0:1, :] = m
        mz_ref[1:2, :] = z

    m = mz_ref[0:1, :]
    z = mz_ref[1:2, :]
    s_blk = s_ref[pl.ds(jb * BS, BS), :]
    w = jnp.exp(s_blk - m) / z
    vwin = jnp.concatenate([vprev_ref[BS - (SPAN - 1):, :], vcur_ref[...]],
                           axis=0)
    accl = jnp.zeros((BS, D // 2), dtype=jnp.float32)
    acch = jnp.zeros((BS, D // 2), dtype=jnp.float32)
    for d in range(SPAN):
        we = w[:, d * H:(d + 1) * H]
        while we.shape[1] < D // 2:
            we = jnp.concatenate([we, we], axis=1)
        vw = vwin[d:d + BS, :]
        accl = accl + we * vw[:, :D // 2]
        acch = acch + we * vw[:, D // 2:]
    acc = jnp.concatenate([accl, acch], axis=1)
    o_ref[...] = jnp.dot(acc, wo_ref[...],
                         preferred_element_type=jnp.float32)


@jax.jit
def kernel(q, k, v, Wq, Wk, Wv, Wo):
    qs = q[0]
    ks = k[0]
    vs = v[0]
    perm = (jnp.arange(D) % H) * DH + jnp.arange(D) // H
    wqT = Wq.T[:, perm] * SCALE
    wkT = Wk.T[:, perm]
    wvT = Wv.T[:, perm]
    woT = Wo.T[perm, :]

    blk = pl.BlockSpec((BS, D), lambda i: (i, 0))
    full_w = pl.BlockSpec((D, D), lambda i: (0, 0))
    prev = pl.BlockSpec((BS, D), lambda i: (jnp.maximum(i - 1, 0), 0))

    Q, K, V = pl.pallas_call(
        _proj_body,
        grid=(NB,),
        in_specs=[blk, blk, blk, full_w, full_w, full_w],
        out_specs=[blk, blk, blk],
        out_shape=[jax.ShapeDtypeStruct((L, D), jnp.float32)] * 3,
    )(qs, ks, vs, wqT, wkT, wvT)

    s_blk = pl.BlockSpec((BS, SPAN * H), lambda i: (i, 0))
    S = pl.pallas_call(
        _score_body,
        grid=(NB,),
        in_specs=[blk, blk, prev],
        out_specs=s_blk,
        out_shape=jax.ShapeDtypeStruct((L, SPAN * H), jnp.float32),
    )(Q, K, K)

    s_full = pl.BlockSpec((L, SPAN * H), lambda i: (0, 0))
    out2d = pl.pallas_call(
        _out_body,
        grid=(NB,),
        in_specs=[s_full, blk, prev, full_w],
        out_specs=blk,
        out_shape=jax.ShapeDtypeStruct((L, D), jnp.float32),
        scratch_shapes=[pltpu.VMEM((8, SPAN * H), jnp.float32)],
    )(S, V, V, woT)

    return out2d[None]


# speedup vs baseline: 13.8106x; 1.2180x over previous
import functools

import jax
import jax.numpy as jnp
from jax.experimental import pallas as pl
from jax.experimental.pallas import tpu as pltpu

H = 12
DH = 64
D = H * DH
SPAN = 32
L = 2048
BS = 512
NB = L // BS
SCALE = 1.0 / (DH ** 0.5)
NEG = -jnp.inf


def _proj_body(q_ref, k_ref, v_ref, wq_ref, wk_ref, wv_ref,
               q_out, k_out, v_out):
    q_out[...] = jnp.dot(q_ref[...], wq_ref[...],
                         preferred_element_type=jnp.float32)
    k_out[...] = jnp.dot(k_ref[...], wk_ref[...],
                         preferred_element_type=jnp.float32)
    v_out[...] = jnp.dot(v_ref[...], wv_ref[...],
                         preferred_element_type=jnp.float32)


DG = 4
W2 = 192


def _score_body(q_ref, kcur_ref, kprev_ref, s_ref):
    jb = pl.program_id(0)
    qb = q_ref[...]
    kwin = jnp.concatenate([kprev_ref[BS - (SPAN - 1):, :], kcur_ref[...]],
                           axis=0)
    rr = jax.lax.broadcasted_iota(jnp.int32, (DG * W2, DG * H), 0)
    cc = jax.lax.broadcasted_iota(jnp.int32, (DG * W2, DG * H), 1)
    rmat = jnp.where((rr % H == cc % H) & (rr // W2 == cc // H),
                     1.0, 0.0).astype(jnp.float32)
    rows = jax.lax.broadcasted_iota(jnp.int32, (BS, DG * H), 0)
    dcol = jax.lax.broadcasted_iota(jnp.int32, (BS, DG * H), 1) // H
    for d0 in range(0, SPAN, DG):
        parts = []
        for d in range(d0, d0 + DG):
            x = qb * kwin[d:d + BS, :]
            x = x[:, :D // 2] + x[:, D // 2:]
            x = x[:, :W2] + x[:, W2:]
            parts.append(x)
        s4 = jnp.dot(jnp.concatenate(parts, axis=1), rmat,
                     preferred_element_type=jnp.float32)
        s4 = jnp.where(jb * BS + rows + (d0 + dcol) - (SPAN - 1) >= 0,
                       s4, NEG)
        s_ref[:, d0 * H:(d0 + DG) * H] = s4


def _out_body(s_ref, vcur_ref, vprev_ref, wo_ref, o_ref, mz_ref):
    jb = pl.program_id(0)

    @pl.when(jb == 0)
    def _():
        s_all = s_ref[...]
        m = jnp.max(s_all, axis=0, keepdims=True)
        z = jnp.sum(jnp.exp(s_all - m), axis=0, keepdims=True)
        mz_ref[0:1, :] = m
        mz_ref[1:2, :] = z

    m = mz_ref[0:1, :]
    z = mz_ref[1:2, :]
    s_blk = s_ref[pl.ds(jb * BS, BS), :]
    w = jnp.exp(s_blk - m) / z
    vwin = jnp.concatenate([vprev_ref[BS - (SPAN - 1):, :], vcur_ref[...]],
                           axis=0)
    accl = jnp.zeros((BS, D // 2), dtype=jnp.float32)
    acch = jnp.zeros((BS, D // 2), dtype=jnp.float32)
    for d in range(SPAN):
        we = w[:, d * H:(d + 1) * H]
        while we.shape[1] < D // 2:
            we = jnp.concatenate([we, we], axis=1)
        vw = vwin[d:d + BS, :]
        accl = accl + we * vw[:, :D // 2]
        acch = acch + we * vw[:, D // 2:]
    acc = jnp.concatenate([accl, acch], axis=1)
    o_ref[...] = jnp.dot(acc, wo_ref[...],
                         preferred_element_type=jnp.float32)


@jax.jit
def kernel(q, k, v, Wq, Wk, Wv, Wo):
    qs = q[0]
    ks = k[0]
    vs = v[0]
    perm = (jnp.arange(D) % H) * DH + jnp.arange(D) // H
    wqT = Wq.T[:, perm] * SCALE
    wkT = Wk.T[:, perm]
    wvT = Wv.T[:, perm]
    woT = Wo.T[perm, :]

    blk = pl.BlockSpec((BS, D), lambda i: (i, 0))
    full_w = pl.BlockSpec((D, D), lambda i: (0, 0))
    prev = pl.BlockSpec((BS, D), lambda i: (jnp.maximum(i - 1, 0), 0))

    Q, K, V = pl.pallas_call(
        _proj_body,
        grid=(NB,),
        in_specs=[blk, blk, blk, full_w, full_w, full_w],
        out_specs=[blk, blk, blk],
        out_shape=[jax.ShapeDtypeStruct((L, D), jnp.float32)] * 3,
    )(qs, ks, vs, wqT, wkT, wvT)

    s_blk = pl.BlockSpec((BS, SPAN * H), lambda i: (i, 0))
    S = pl.pallas_call(
        _score_body,
        grid=(NB,),
        in_specs=[blk, blk, prev],
        out_specs=s_blk,
        out_shape=jax.ShapeDtypeStruct((L, SPAN * H), jnp.float32),
    )(Q, K, K)

    s_full = pl.BlockSpec((L, SPAN * H), lambda i: (0, 0))
    out2d = pl.pallas_call(
        _out_body,
        grid=(NB,),
        in_specs=[s_full, blk, prev, full_w],
        out_specs=blk,
        out_shape=jax.ShapeDtypeStruct((L, D), jnp.float32),
        scratch_shapes=[pltpu.VMEM((8, SPAN * H), jnp.float32)],
    )(S, V, V, woT)

    return out2d[None]


# speedup vs baseline: 15.1438x; 1.0965x over previous
import jax
import jax.numpy as jnp
from jax.experimental import pallas as pl
from jax.experimental.pallas import tpu as pltpu

H = 12
DH = 64
D = H * DH
SPAN = 32
L = 2048
BS = 512
NB = L // BS
SCALE = 1.0 / (DH ** 0.5)
NEG = -jnp.inf
DG = 4
W2 = 192


def _fused_body(q_ref, k_ref, v_ref, wq_ref, wk_ref, wv_ref, wo_ref,
                o_ref, ks_ref, vs_ref, ss_ref, mz_ref):
    ph = pl.program_id(0)
    jb = pl.program_id(1)

    @pl.when(ph == 0)
    def _proj_scores():
        @pl.when(jb == 0)
        def _():
            ks_ref[0:SPAN, :] = jnp.zeros((SPAN, D), jnp.float32)
            vs_ref[0:SPAN, :] = jnp.zeros((SPAN, D), jnp.float32)

        qb = jnp.dot(
            q_ref[...], wq_ref[...], preferred_element_type=jnp.float32)
        ks_ref[pl.ds(jb * BS + SPAN, BS), :] = jnp.dot(
            k_ref[...], wk_ref[...], preferred_element_type=jnp.float32)
        vs_ref[pl.ds(jb * BS + SPAN, BS), :] = jnp.dot(
            v_ref[...], wv_ref[...], preferred_element_type=jnp.float32)
        kwin = ks_ref[pl.ds(jb * BS, BS + SPAN), :]
        rr = jax.lax.broadcasted_iota(jnp.int32, (DG * W2, DG * H), 0)
        cc = jax.lax.broadcasted_iota(jnp.int32, (DG * W2, DG * H), 1)
        rmat = jnp.where((rr % H == cc % H) & (rr // W2 == cc // H),
                         1.0, 0.0).astype(jnp.float32)
        rows = jax.lax.broadcasted_iota(jnp.int32, (BS, DG * H), 0)
        dcol = jax.lax.broadcasted_iota(jnp.int32, (BS, DG * H), 1) // H
        for d0 in range(0, SPAN, DG):
            parts = []
            for d in range(d0, d0 + DG):
                x = qb * kwin[d + 1:d + 1 + BS, :]
                x = x[:, :D // 2] + x[:, D // 2:]
                x = x[:, :W2] + x[:, W2:]
                parts.append(x)
            s4 = jnp.dot(jnp.concatenate(parts, axis=1), rmat,
                         preferred_element_type=jnp.float32)
            s4 = jnp.where(jb * BS + rows + (d0 + dcol) - (SPAN - 1) >= 0,
                           s4, NEG)
            ss_ref[pl.ds(jb * BS, BS), d0 * H:(d0 + DG) * H] = s4

    @pl.when(ph == 1)
    def _out():
        @pl.when(jb == 0)
        def _():
            s_all = ss_ref[...]
            m = jnp.max(s_all, axis=0, keepdims=True)
            z = jnp.sum(jnp.exp(s_all - m), axis=0, keepdims=True)
            mz_ref[0:1, :] = m
            mz_ref[1:2, :] = z

        m = mz_ref[0:1, :]
        z = mz_ref[1:2, :]
        s_blk = ss_ref[pl.ds(jb * BS, BS), :]
        w = jnp.exp(s_blk - m) / z
        vwin = vs_ref[pl.ds(jb * BS, BS + SPAN), :]
        accl = jnp.zeros((BS, D // 2), dtype=jnp.float32)
        acch = jnp.zeros((BS, D // 2), dtype=jnp.float32)
        for d in range(SPAN):
            we = w[:, d * H:(d + 1) * H]
            while we.shape[1] < D // 2:
                we = jnp.concatenate([we, we], axis=1)
            vw = vwin[d + 1:d + 1 + BS, :]
            accl = accl + we * vw[:, :D // 2]
            acch = acch + we * vw[:, D // 2:]
        acc = jnp.concatenate([accl, acch], axis=1)
        o_ref[...] = jnp.dot(acc, wo_ref[...],
                             preferred_element_type=jnp.float32)


@jax.jit
def kernel(q, k, v, Wq, Wk, Wv, Wo):
    qs = q[0]
    ks = k[0]
    vs = v[0]
    perm = (jnp.arange(D) % H) * DH + jnp.arange(D) // H
    wqT = Wq.T[:, perm] * SCALE
    wkT = Wk.T[:, perm]
    wvT = Wv.T[:, perm]
    woT = Wo.T[perm, :]

    inp = pl.BlockSpec((BS, D), lambda p, i: (jnp.where(p == 0, i, 0), 0))
    full_w = pl.BlockSpec((D, D), lambda p, i: (0, 0))
    outs = pl.BlockSpec((BS, D), lambda p, i: (i, 0))

    out2d = pl.pallas_call(
        _fused_body,
        grid=(2, NB),
        in_specs=[inp, inp, inp, full_w, full_w, full_w, full_w],
        out_specs=outs,
        out_shape=jax.ShapeDtypeStruct((L, D), jnp.float32),
        compiler_params=pltpu.CompilerParams(
            vmem_limit_bytes=100 * 1024 * 1024),
        scratch_shapes=[
            pltpu.VMEM((L + SPAN, D), jnp.float32),
            pltpu.VMEM((L + SPAN, D), jnp.float32),
            pltpu.VMEM((L, SPAN * H), jnp.float32),
            pltpu.VMEM((8, SPAN * H), jnp.float32),
        ],
    )(qs, ks, vs, wqT, wkT, wvT, woT)

    return out2d[None]
